# baseline (speedup 1.0000x reference)
import functools

import jax
import jax.numpy as jnp
from jax import lax
from jax.experimental import pallas as pl
from jax.experimental.pallas import tpu as pltpu
from jax.experimental.pallas import tpu_sc as plsc

N = 10000
E = 320000
D = 128
NC = 2
NS = 16
NW = NC * NS
K = 256
NB = 40
EPT = NB * K
EPAD = NW * EPT
NACC = 10240
RPT = 624
RPT_LAST = N - (NS - 1) * RPT
DEGZ = NACC // NS

_mesh = lambda: plsc.VectorSubcoreMesh(core_axis_name="c", subcore_axis_name="s")


def _deg_body(dstb_hbm, deg_hbm, dst_v, ones_v, zer_v, dacc):
  c = lax.axis_index("c")
  s = lax.axis_index("s")
  wid = c * NS + s
  pltpu.sync_copy(dstb_hbm.at[wid], dst_v)
  for j in range(K // 32):
    ones_v[pl.ds(j * 16, 16)] = jnp.ones((16,), jnp.float32)
  for j in range(DEGZ // 16):
    zer_v[pl.ds(j * 16, 16)] = jnp.zeros((16,), jnp.float32)
  pltpu.sync_copy(zer_v, dacc.at[pl.ds(s * DEGZ, DEGZ)])
  plsc.subcore_barrier()

  @pl.loop(0, NB * 2)
  def _edge_batches(b):
    pltpu.sync_copy(ones_v, dacc.at[dst_v.at[b]], add=True)

  plsc.subcore_barrier()
  pltpu.sync_copy(dacc.at[pl.ds(s * DEGZ, DEGZ)],
                  deg_hbm.at[c, 0, pl.ds(s * DEGZ, DEGZ)])


def _deg_call(dstb):
  f = pl.kernel(
      _deg_body,
      out_type=jax.ShapeDtypeStruct((NC, 1, NACC), jnp.float32),
      mesh=_mesh(),
      scratch_types=[
          pltpu.VMEM((NB * 2, K // 2), jnp.int32),
          pltpu.VMEM((K // 2,), jnp.float32),
          pltpu.VMEM((DEGZ,), jnp.float32),
          pltpu.VMEM_SHARED((NACC,), jnp.float32),
      ],
  )
  return f(dstb)


def _edge_body(m_hbm, srcb_hbm, dstb_hbm, zrows_hbm, out_hbm,
               sidx0, sidx1, dst_v, buf, acc, gsem, isem0, isem1):
  c = lax.axis_index("c")
  s = lax.axis_index("s")
  wid = c * NS + s
  pltpu.sync_copy(dstb_hbm.at[wid], dst_v)
  rbase = s * RPT
  last = s == NS - 1

  @pl.when(jnp.logical_and(c == 0, jnp.logical_not(last)))
  def _():
    pltpu.sync_copy(m_hbm.at[pl.ds(rbase, RPT)], acc.at[pl.ds(rbase, RPT)])

  @pl.when(jnp.logical_and(c == 0, last))
  def _():
    pltpu.sync_copy(m_hbm.at[pl.ds((NS - 1) * RPT, RPT_LAST)],
                    acc.at[pl.ds((NS - 1) * RPT, RPT_LAST)])

  @pl.when(jnp.logical_and(c != 0, jnp.logical_not(last)))
  def _():
    pltpu.sync_copy(zrows_hbm.at[pl.ds(0, RPT)], acc.at[pl.ds(rbase, RPT)])

  @pl.when(jnp.logical_and(c != 0, last))
  def _():
    pltpu.sync_copy(zrows_hbm, acc.at[pl.ds((NS - 1) * RPT, RPT_LAST)])

  plsc.subcore_barrier()

  sidx = (sidx0, sidx1)
  isem = (isem0, isem1)

  def wait_idx(j):
    pltpu.make_async_copy(srcb_hbm.at[0, 0], sidx[j], isem[j]).wait()

  pltpu.async_copy(srcb_hbm.at[wid, 0], sidx0, isem0)
  pltpu.async_copy(srcb_hbm.at[wid, 1], sidx1, isem1)

  @pl.loop(0, NB // 2)
  def _groups(o):
    B = 2 * o
    for k in range(2):
      b = B + k
      wait_idx(k)
      pltpu.async_copy(m_hbm.at[sidx[k].at[0]], buf, gsem).wait()

      @pl.when(b + 2 < NB)
      def _():
        pltpu.async_copy(srcb_hbm.at[wid, b + 2], sidx[k], isem[k])

      pltpu.sync_copy(buf.at[pl.ds(0, K // 2)],
                      acc.at[dst_v.at[2 * b]], add=True)
      pltpu.sync_copy(buf.at[pl.ds(K // 2, K // 2)],
                      acc.at[dst_v.at[2 * b + 1]], add=True)

  plsc.subcore_barrier()

  @pl.when(jnp.logical_not(last))
  def _():
    pltpu.sync_copy(acc.at[pl.ds(rbase, RPT)],
                    out_hbm.at[c, pl.ds(rbase, RPT)])

  @pl.when(last)
  def _():
    pltpu.sync_copy(acc.at[pl.ds((NS - 1) * RPT, RPT_LAST)],
                    out_hbm.at[c, pl.ds((NS - 1) * RPT, RPT_LAST)])


def _edge_call(m, srcb4, dstb, zrows):
  f = pl.kernel(
      _edge_body,
      out_type=jax.ShapeDtypeStruct((NC, N, D), jnp.float32),
      mesh=_mesh(),
      scratch_types=(
          [pltpu.VMEM((1, K), jnp.int32)] * 2
          + [pltpu.VMEM((NB * 2, K // 2), jnp.int32)]
          + [pltpu.VMEM((K, D), jnp.float32)]
          + [pltpu.VMEM_SHARED((NACC, D), jnp.float32)]
          + [pltpu.SemaphoreType.DMA] * 3
      ),
  )
  return f(m, srcb4, dstb, zrows)


RB = 1000


def _tc_first_body(x_ref, wx_ref, bx_ref, w0_ref, dinv_ref, o_ref):
  h0 = jnp.dot(x_ref[...], wx_ref[...],
               preferred_element_type=jnp.float32) + bx_ref[...]
  o_ref[...] = jnp.dot(h0, w0_ref[...],
                       preferred_element_type=jnp.float32) * dinv_ref[...]


def _tc_mid_body(s_ref, b_ref, w_ref, dinv_ref, o_ref):
  h = jnp.maximum(dinv_ref[...] * (s_ref[0] + s_ref[1]) + b_ref[...], 0.0)
  o_ref[...] = jnp.dot(h, w_ref[...],
                       preferred_element_type=jnp.float32) * dinv_ref[...]


def _tc_last_body(s_ref, b_ref, dinv_ref, o_ref):
  o_ref[...] = dinv_ref[...] * (s_ref[0] + s_ref[1]) + b_ref[...]


_row_spec = pl.BlockSpec((RB, D), lambda i: (i, 0))
_w_spec = pl.BlockSpec((D, D), lambda i: (0, 0))
_b_spec = pl.BlockSpec((1, D), lambda i: (0, 0))
_dinv_spec = pl.BlockSpec((RB, 1), lambda i: (i, 0))
_s_spec = pl.BlockSpec((NC, RB, D), lambda i: (0, i, 0))
_out_sds = jax.ShapeDtypeStruct((N, D), jnp.float32)


def _tc_first(x, Wx, bx2, W0, dinv):
  return pl.pallas_call(
      _tc_first_body, grid=(N // RB,),
      in_specs=[_row_spec, _w_spec, _b_spec, _w_spec, _dinv_spec],
      out_specs=_row_spec, out_shape=_out_sds,
  )(x, Wx, bx2, W0, dinv)


def _tc_mid(s2, b2, W, dinv):
  return pl.pallas_call(
      _tc_mid_body, grid=(N // RB,),
      in_specs=[_s_spec, _b_spec, _w_spec, _dinv_spec],
      out_specs=_row_spec, out_shape=_out_sds,
  )(s2, b2, W, dinv)


def _tc_last(s2, b2, dinv):
  return pl.pallas_call(
      _tc_last_body, grid=(N // RB,),
      in_specs=[_s_spec, _b_spec, _dinv_spec],
      out_specs=_row_spec, out_shape=_out_sds,
  )(s2, b2, dinv)


def kernel(x, edge_index, edge_attr, Wx, bx, W0, b0, W1, b1, W2, b2):
  del edge_attr
  src = edge_index[0].astype(jnp.int32)
  dst = edge_index[1].astype(jnp.int32)
  pad = EPAD - E
  ar = jnp.arange(pad, dtype=jnp.int32)
  pad_src = (ar * 97) % N
  pad_dst = N + ar % (NACC - N)
  srcb4 = jnp.concatenate([src, pad_src]).reshape(NW, NB, 1, K)
  dstb = jnp.concatenate([dst, pad_dst]).reshape(NW, NB * 2, K // 2)
  zrows = jnp.zeros((RPT_LAST, D), jnp.float32)

  deg2 = _deg_call(dstb)
  deg = deg2[0, 0, :N] + deg2[1, 0, :N] + 1.0
  dinv = lax.rsqrt(deg).reshape(N, 1)

  bx2 = bx.reshape(1, D)
  m = _tc_first(x, Wx, bx2, W0, dinv)
  s2 = _edge_call(m, srcb4, dstb, zrows)
  m = _tc_mid(s2, b0.reshape(1, D), W1, dinv)
  s2 = _edge_call(m, srcb4, dstb, zrows)
  m = _tc_mid(s2, b1.reshape(1, D), W2, dinv)
  s2 = _edge_call(m, srcb4, dstb, zrows)
  return _tc_last(s2, b2.reshape(1, D), dinv)


# speedup vs baseline: 1.2899x; 1.2899x over previous
import functools

import jax
import jax.numpy as jnp
from jax import lax
from jax.experimental import pallas as pl
from jax.experimental.pallas import tpu as pltpu
from jax.experimental.pallas import tpu_sc as plsc

N = 10000
E = 320000
D = 128
NC = 2
NS = 16
NW = NC * NS
K = 128
NB = 80
EPT = NB * K
EPAD = NW * EPT
NACC = 10240
RPT = 624
RPT_LAST = N - (NS - 1) * RPT
DEGZ = NACC // NS

_mesh = lambda: plsc.VectorSubcoreMesh(core_axis_name="c", subcore_axis_name="s")


def _deg_body(dstb_hbm, deg_hbm, dst_v, ones_v, zer_v, dacc):
  c = lax.axis_index("c")
  s = lax.axis_index("s")
  wid = c * NS + s
  pltpu.sync_copy(dstb_hbm.at[wid], dst_v)
  for j in range(K // 16):
    ones_v[pl.ds(j * 16, 16)] = jnp.ones((16,), jnp.float32)
  for j in range(DEGZ // 16):
    zer_v[pl.ds(j * 16, 16)] = jnp.zeros((16,), jnp.float32)
  pltpu.sync_copy(zer_v, dacc.at[pl.ds(s * DEGZ, DEGZ)])
  plsc.subcore_barrier()

  @pl.loop(0, NB)
  def _edge_batches(b):
    pltpu.sync_copy(ones_v, dacc.at[dst_v.at[b]], add=True)

  plsc.subcore_barrier()
  pltpu.sync_copy(dacc.at[pl.ds(s * DEGZ, DEGZ)],
                  deg_hbm.at[c, 0, pl.ds(s * DEGZ, DEGZ)])


def _deg_call(dstb):
  f = pl.kernel(
      _deg_body,
      out_type=jax.ShapeDtypeStruct((NC, 1, NACC), jnp.float32),
      mesh=_mesh(),
      scratch_types=[
          pltpu.VMEM((NB, K), jnp.int32),
          pltpu.VMEM((K,), jnp.float32),
          pltpu.VMEM((DEGZ,), jnp.float32),
          pltpu.VMEM_SHARED((NACC,), jnp.float32),
      ],
  )
  return f(dstb)


def _edge_body(m_hbm, srcb_hbm, dstb_hbm, zrows_hbm, out_hbm,
               sidx0, sidx1, dst_v, buf0, buf1, acc,
               sem0, sem1, isem0, isem1):
  c = lax.axis_index("c")
  s = lax.axis_index("s")
  wid = c * NS + s
  pltpu.sync_copy(dstb_hbm.at[wid], dst_v)
  rbase = s * RPT
  last = s == NS - 1

  @pl.when(jnp.logical_and(c == 0, jnp.logical_not(last)))
  def _():
    pltpu.sync_copy(m_hbm.at[pl.ds(rbase, RPT)], acc.at[pl.ds(rbase, RPT)])

  @pl.when(jnp.logical_and(c == 0, last))
  def _():
    pltpu.sync_copy(m_hbm.at[pl.ds((NS - 1) * RPT, RPT_LAST)],
                    acc.at[pl.ds((NS - 1) * RPT, RPT_LAST)])

  @pl.when(jnp.logical_and(c != 0, jnp.logical_not(last)))
  def _():
    pltpu.sync_copy(zrows_hbm.at[pl.ds(0, RPT)], acc.at[pl.ds(rbase, RPT)])

  @pl.when(jnp.logical_and(c != 0, last))
  def _():
    pltpu.sync_copy(zrows_hbm, acc.at[pl.ds((NS - 1) * RPT, RPT_LAST)])

  plsc.subcore_barrier()

  def wait_idx(sref, isem):
    pltpu.make_async_copy(srcb_hbm.at[0, 0], sref, isem).wait()

  def wait_gather(bref, sem):
    pltpu.make_async_copy(m_hbm.at[sidx0.at[0]], bref, sem).wait()

  pltpu.async_copy(srcb_hbm.at[wid, 0], sidx0, isem0)
  pltpu.async_copy(srcb_hbm.at[wid, 1], sidx1, isem1)
  wait_idx(sidx0, isem0)
  pltpu.async_copy(m_hbm.at[sidx0.at[0]], buf0, sem0)

  @pl.loop(0, NB // 2)
  def _groups(o):
    b0 = 2 * o
    b1 = b0 + 1
    wait_idx(sidx1, isem1)
    pltpu.async_copy(m_hbm.at[sidx1.at[0]], buf1, sem1)
    wait_gather(buf0, sem0)

    @pl.when(b0 + 2 < NB)
    def _():
      pltpu.async_copy(srcb_hbm.at[wid, b0 + 2], sidx0, isem0)

    pltpu.sync_copy(buf0, acc.at[dst_v.at[b0]], add=True)
    wait_gather(buf1, sem1)

    @pl.when(b1 + 2 < NB)
    def _():
      pltpu.async_copy(srcb_hbm.at[wid, b1 + 2], sidx1, isem1)

    @pl.when(b0 + 2 < NB)
    def _():
      wait_idx(sidx0, isem0)
      pltpu.async_copy(m_hbm.at[sidx0.at[0]], buf0, sem0)

    pltpu.sync_copy(buf1, acc.at[dst_v.at[b1]], add=True)

  plsc.subcore_barrier()

  @pl.when(jnp.logical_not(last))
  def _():
    pltpu.sync_copy(acc.at[pl.ds(rbase, RPT)],
                    out_hbm.at[c, pl.ds(rbase, RPT)])

  @pl.when(last)
  def _():
    pltpu.sync_copy(acc.at[pl.ds((NS - 1) * RPT, RPT_LAST)],
                    out_hbm.at[c, pl.ds((NS - 1) * RPT, RPT_LAST)])


def _edge_call(m, srcb4, dstb, zrows):
  f = pl.kernel(
      _edge_body,
      out_type=jax.ShapeDtypeStruct((NC, N, D), jnp.float32),
      mesh=_mesh(),
      scratch_types=(
          [pltpu.VMEM((1, K), jnp.int32)] * 2
          + [pltpu.VMEM((NB, K), jnp.int32)]
          + [pltpu.VMEM((K, D), jnp.float32)] * 2
          + [pltpu.VMEM_SHARED((NACC, D), jnp.float32)]
          + [pltpu.SemaphoreType.DMA] * 4
      ),
  )
  return f(m, srcb4, dstb, zrows)


RB = 1000


def _tc_first_body(x_ref, wx_ref, bx_ref, w0_ref, dinv_ref, o_ref):
  h0 = jnp.dot(x_ref[...], wx_ref[...],
               preferred_element_type=jnp.float32) + bx_ref[...]
  o_ref[...] = jnp.dot(h0, w0_ref[...],
                       preferred_element_type=jnp.float32) * dinv_ref[...]


def _tc_mid_body(s_ref, b_ref, w_ref, dinv_ref, o_ref):
  h = jnp.maximum(dinv_ref[...] * (s_ref[0] + s_ref[1]) + b_ref[...], 0.0)
  o_ref[...] = jnp.dot(h, w_ref[...],
                       preferred_element_type=jnp.float32) * dinv_ref[...]


def _tc_last_body(s_ref, b_ref, dinv_ref, o_ref):
  o_ref[...] = dinv_ref[...] * (s_ref[0] + s_ref[1]) + b_ref[...]


_row_spec = pl.BlockSpec((RB, D), lambda i: (i, 0))
_w_spec = pl.BlockSpec((D, D), lambda i: (0, 0))
_b_spec = pl.BlockSpec((1, D), lambda i: (0, 0))
_dinv_spec = pl.BlockSpec((RB, 1), lambda i: (i, 0))
_s_spec = pl.BlockSpec((NC, RB, D), lambda i: (0, i, 0))
_out_sds = jax.ShapeDtypeStruct((N, D), jnp.float32)


def _tc_first(x, Wx, bx2, W0, dinv):
  return pl.pallas_call(
      _tc_first_body, grid=(N // RB,),
      in_specs=[_row_spec, _w_spec, _b_spec, _w_spec, _dinv_spec],
      out_specs=_row_spec, out_shape=_out_sds,
  )(x, Wx, bx2, W0, dinv)


def _tc_mid(s2, b2, W, dinv):
  return pl.pallas_call(
      _tc_mid_body, grid=(N // RB,),
      in_specs=[_s_spec, _b_spec, _w_spec, _dinv_spec],
      out_specs=_row_spec, out_shape=_out_sds,
  )(s2, b2, W, dinv)


def _tc_last(s2, b2, dinv):
  return pl.pallas_call(
      _tc_last_body, grid=(N // RB,),
      in_specs=[_s_spec, _b_spec, _dinv_spec],
      out_specs=_row_spec, out_shape=_out_sds,
  )(s2, b2, dinv)


def kernel(x, edge_index, edge_attr, Wx, bx, W0, b0, W1, b1, W2, b2):
  del edge_attr
  src = edge_index[0].astype(jnp.int32)
  dst = edge_index[1].astype(jnp.int32)
  pad = EPAD - E
  ar = jnp.arange(pad, dtype=jnp.int32)
  pad_src = (ar * 97) % N
  pad_dst = N + ar % (NACC - N)
  srcb4 = jnp.concatenate([src, pad_src]).reshape(NW, NB, 1, K)
  dstb = jnp.concatenate([dst, pad_dst]).reshape(NW, NB, K)
  zrows = jnp.zeros((RPT_LAST, D), jnp.float32)

  deg2 = _deg_call(dstb)
  deg = deg2[0, 0, :N] + deg2[1, 0, :N] + 1.0
  dinv = lax.rsqrt(deg).reshape(N, 1)

  bx2 = bx.reshape(1, D)
  m = _tc_first(x, Wx, bx2, W0, dinv)
  s2 = _edge_call(m, srcb4, dstb, zrows)
  m = _tc_mid(s2, b0.reshape(1, D), W1, dinv)
  s2 = _edge_call(m, srcb4, dstb, zrows)
  m = _tc_mid(s2, b1.reshape(1, D), W2, dinv)
  s2 = _edge_call(m, srcb4, dstb, zrows)
  return _tc_last(s2, b2.reshape(1, D), dinv)


# speedup vs baseline: 1.4550x; 1.1279x over previous
import functools

import jax
import jax.numpy as jnp
from jax import lax
from jax.experimental import pallas as pl
from jax.experimental.pallas import tpu as pltpu
from jax.experimental.pallas import tpu_sc as plsc

N = 10000
E = 320000
D = 128
NC = 2
NS = 16
NW = NC * NS
K = 128
NB = 81
EPT = NB * K
EPAD = NW * EPT
NACC = 10104
DEGN = 10240
RPT = 624
RPT_LAST = N - (NS - 1) * RPT
DEGZ = DEGN // NS

_mesh = lambda: plsc.VectorSubcoreMesh(core_axis_name="c", subcore_axis_name="s")


def _deg_body(dstb_hbm, deg_hbm, dst_v, ones_v, zer_v, dacc):
  c = lax.axis_index("c")
  s = lax.axis_index("s")
  wid = c * NS + s
  pltpu.sync_copy(dstb_hbm.at[wid], dst_v)
  for j in range(K // 16):
    ones_v[pl.ds(j * 16, 16)] = jnp.ones((16,), jnp.float32)
  for j in range(DEGZ // 16):
    zer_v[pl.ds(j * 16, 16)] = jnp.zeros((16,), jnp.float32)
  pltpu.sync_copy(zer_v, dacc.at[pl.ds(s * DEGZ, DEGZ)])
  plsc.subcore_barrier()

  @pl.loop(0, NB)
  def _edge_batches(b):
    pltpu.sync_copy(ones_v, dacc.at[dst_v.at[b]], add=True)

  plsc.subcore_barrier()
  pltpu.sync_copy(dacc.at[pl.ds(s * DEGZ, DEGZ)],
                  deg_hbm.at[c, 0, pl.ds(s * DEGZ, DEGZ)])


def _deg_call(dstb):
  f = pl.kernel(
      _deg_body,
      out_type=jax.ShapeDtypeStruct((NC, 1, DEGN), jnp.float32),
      mesh=_mesh(),
      scratch_types=[
          pltpu.VMEM((NB, K), jnp.int32),
          pltpu.VMEM((K,), jnp.float32),
          pltpu.VMEM((DEGZ,), jnp.float32),
          pltpu.VMEM_SHARED((DEGN,), jnp.float32),
      ],
  )
  return f(dstb)


def _edge_body(m_hbm, srcb_hbm, dstb_hbm, zrows_hbm, out_hbm,
               sidx0, sidx1, sidx2, didx0, didx1, didx2,
               buf0, buf1, buf2, acc,
               gsem0, gsem1, gsem2, isem0, isem1, isem2,
               dsem0, dsem1, dsem2):
  c = lax.axis_index("c")
  s = lax.axis_index("s")
  wid = c * NS + s
  rbase = s * RPT
  last = s == NS - 1

  @pl.when(jnp.logical_and(c == 0, jnp.logical_not(last)))
  def _():
    pltpu.sync_copy(m_hbm.at[pl.ds(rbase, RPT)], acc.at[pl.ds(rbase, RPT)])

  @pl.when(jnp.logical_and(c == 0, last))
  def _():
    pltpu.sync_copy(m_hbm.at[pl.ds((NS - 1) * RPT, RPT_LAST)],
                    acc.at[pl.ds((NS - 1) * RPT, RPT_LAST)])

  @pl.when(jnp.logical_and(c != 0, jnp.logical_not(last)))
  def _():
    pltpu.sync_copy(zrows_hbm.at[pl.ds(0, RPT)], acc.at[pl.ds(rbase, RPT)])

  @pl.when(jnp.logical_and(c != 0, last))
  def _():
    pltpu.sync_copy(zrows_hbm, acc.at[pl.ds((NS - 1) * RPT, RPT_LAST)])

  plsc.subcore_barrier()

  sidx = (sidx0, sidx1, sidx2)
  didx = (didx0, didx1, didx2)
  bufs = (buf0, buf1, buf2)
  gsem = (gsem0, gsem1, gsem2)
  isem = (isem0, isem1, isem2)
  dsem = (dsem0, dsem1, dsem2)

  def wait_sidx(j):
    pltpu.make_async_copy(srcb_hbm.at[0, 0], sidx[j], isem[j]).wait()

  def wait_didx(j):
    pltpu.make_async_copy(dstb_hbm.at[0, 0], didx[j], dsem[j]).wait()

  def wait_gather(j):
    pltpu.make_async_copy(m_hbm.at[sidx0.at[0]], bufs[j], gsem[j]).wait()

  for j in range(3):
    pltpu.async_copy(srcb_hbm.at[wid, j], sidx[j], isem[j])
    pltpu.async_copy(dstb_hbm.at[wid, j], didx[j], dsem[j])
  for j in range(3):
    wait_sidx(j)
    pltpu.async_copy(m_hbm.at[sidx[j].at[0]], bufs[j], gsem[j])

  @pl.loop(0, NB // 3)
  def _groups(o):
    B = 3 * o
    for j in range(3):
      b = B + j
      wait_gather(j)

      @pl.when(b + 3 < NB)
      def _():
        pltpu.async_copy(srcb_hbm.at[wid, b + 3], sidx[j], isem[j])

      wait_didx(j)
      pltpu.sync_copy(bufs[j], acc.at[didx[j].at[0]], add=True)

      @pl.when(b + 3 < NB)
      def _():
        pltpu.async_copy(dstb_hbm.at[wid, b + 3], didx[j], dsem[j])
        wait_sidx(j)
        pltpu.async_copy(m_hbm.at[sidx[j].at[0]], bufs[j], gsem[j])

  plsc.subcore_barrier()

  @pl.when(jnp.logical_not(last))
  def _():
    pltpu.sync_copy(acc.at[pl.ds(rbase, RPT)],
                    out_hbm.at[c, pl.ds(rbase, RPT)])

  @pl.when(last)
  def _():
    pltpu.sync_copy(acc.at[pl.ds((NS - 1) * RPT, RPT_LAST)],
                    out_hbm.at[c, pl.ds((NS - 1) * RPT, RPT_LAST)])


def _edge_call(m, srcb4, dstb4, zrows):
  f = pl.kernel(
      _edge_body,
      out_type=jax.ShapeDtypeStruct((NC, N, D), jnp.float32),
      mesh=_mesh(),
      scratch_types=(
          [pltpu.VMEM((1, K), jnp.int32)] * 6
          + [pltpu.VMEM((K, D), jnp.float32)] * 3
          + [pltpu.VMEM_SHARED((NACC, D), jnp.float32)]
          + [pltpu.SemaphoreType.DMA] * 9
      ),
  )
  return f(m, srcb4, dstb4, zrows)


RB = 1000


def _tc_first_body(x_ref, wx_ref, bx_ref, w0_ref, dinv_ref, o_ref):
  h0 = jnp.dot(x_ref[...], wx_ref[...],
               preferred_element_type=jnp.float32) + bx_ref[...]
  o_ref[...] = jnp.dot(h0, w0_ref[...],
                       preferred_element_type=jnp.float32) * dinv_ref[...]


def _tc_mid_body(s_ref, b_ref, w_ref, dinv_ref, o_ref):
  h = jnp.maximum(dinv_ref[...] * (s_ref[0] + s_ref[1]) + b_ref[...], 0.0)
  o_ref[...] = jnp.dot(h, w_ref[...],
                       preferred_element_type=jnp.float32) * dinv_ref[...]


def _tc_last_body(s_ref, b_ref, dinv_ref, o_ref):
  o_ref[...] = dinv_ref[...] * (s_ref[0] + s_ref[1]) + b_ref[...]


_row_spec = pl.BlockSpec((RB, D), lambda i: (i, 0))
_w_spec = pl.BlockSpec((D, D), lambda i: (0, 0))
_b_spec = pl.BlockSpec((1, D), lambda i: (0, 0))
_dinv_spec = pl.BlockSpec((RB, 1), lambda i: (i, 0))
_s_spec = pl.BlockSpec((NC, RB, D), lambda i: (0, i, 0))
_out_sds = jax.ShapeDtypeStruct((N, D), jnp.float32)


def _tc_first(x, Wx, bx2, W0, dinv):
  return pl.pallas_call(
      _tc_first_body, grid=(N // RB,),
      in_specs=[_row_spec, _w_spec, _b_spec, _w_spec, _dinv_spec],
      out_specs=_row_spec, out_shape=_out_sds,
  )(x, Wx, bx2, W0, dinv)


def _tc_mid(s2, b2, W, dinv):
  return pl.pallas_call(
      _tc_mid_body, grid=(N // RB,),
      in_specs=[_s_spec, _b_spec, _w_spec, _dinv_spec],
      out_specs=_row_spec, out_shape=_out_sds,
  )(s2, b2, W, dinv)


def _tc_last(s2, b2, dinv):
  return pl.pallas_call(
      _tc_last_body, grid=(N // RB,),
      in_specs=[_s_spec, _b_spec, _dinv_spec],
      out_specs=_row_spec, out_shape=_out_sds,
  )(s2, b2, dinv)


def kernel(x, edge_index, edge_attr, Wx, bx, W0, b0, W1, b1, W2, b2):
  del edge_attr
  src = edge_index[0].astype(jnp.int32)
  dst = edge_index[1].astype(jnp.int32)
  pad = EPAD - E
  ar = jnp.arange(pad, dtype=jnp.int32)
  pad_src = (ar * 97) % N
  pad_dst = N + ar % (NACC - N)
  srcb4 = jnp.concatenate([src, pad_src]).reshape(NW, NB, 1, K)
  dstb4 = jnp.concatenate([dst, pad_dst]).reshape(NW, NB, 1, K)
  zrows = jnp.zeros((RPT_LAST, D), jnp.float32)

  deg2 = _deg_call(dstb4.reshape(NW, NB, K))
  deg = deg2[0, 0, :N] + deg2[1, 0, :N] + 1.0
  dinv = lax.rsqrt(deg).reshape(N, 1)

  bx2 = bx.reshape(1, D)
  m = _tc_first(x, Wx, bx2, W0, dinv)
  s2 = _edge_call(m, srcb4, dstb4, zrows)
  m = _tc_mid(s2, b0.reshape(1, D), W1, dinv)
  s2 = _edge_call(m, srcb4, dstb4, zrows)
  m = _tc_mid(s2, b1.reshape(1, D), W2, dinv)
  s2 = _edge_call(m, srcb4, dstb4, zrows)
  return _tc_last(s2, b2.reshape(1, D), dinv)


# speedup vs baseline: 1.4595x; 1.0032x over previous
import jax
import jax.numpy as jnp
from jax import lax
from jax.experimental import pallas as pl
from jax.experimental.pallas import tpu as pltpu
from jax.experimental.pallas import tpu_sc as plsc

N = 10000
E = 320000
D = 128
NC = 2
NS = 16
NW = NC * NS
K = 128
NB = 81
EPT = NB * K
EPAD = NW * EPT
NACC = 10104
DEGN = 10240
RPT = 624
RPT_LAST = N - (NS - 1) * RPT
DEGZ = DEGN // NS

_mesh = lambda: plsc.VectorSubcoreMesh(core_axis_name="c", subcore_axis_name="s")


def _deg_body(dstb_hbm, deg_hbm, dst_v, ones_v, zer_v, dacc):
  c = lax.axis_index("c")
  s = lax.axis_index("s")
  wid = c * NS + s
  pltpu.sync_copy(dstb_hbm.at[wid], dst_v)
  for j in range(K // 16):
    ones_v[pl.ds(j * 16, 16)] = jnp.ones((16,), jnp.float32)
  for j in range(DEGZ // 16):
    zer_v[pl.ds(j * 16, 16)] = jnp.zeros((16,), jnp.float32)
  pltpu.sync_copy(zer_v, dacc.at[pl.ds(s * DEGZ, DEGZ)])
  plsc.subcore_barrier()

  @pl.loop(0, NB)
  def _edge_batches(b):
    pltpu.sync_copy(ones_v, dacc.at[dst_v.at[b]], add=True)

  plsc.subcore_barrier()
  pltpu.sync_copy(dacc.at[pl.ds(s * DEGZ, DEGZ)],
                  deg_hbm.at[c, 0, pl.ds(s * DEGZ, DEGZ)])


def _deg_call(dstb):
  f = pl.kernel(
      _deg_body,
      out_type=jax.ShapeDtypeStruct((NC, 1, DEGN), jnp.float32),
      mesh=_mesh(),
      scratch_types=[
          pltpu.VMEM((NB, K), jnp.int32),
          pltpu.VMEM((K,), jnp.float32),
          pltpu.VMEM((DEGZ,), jnp.float32),
          pltpu.VMEM_SHARED((DEGN,), jnp.float32),
      ],
  )
  return f(dstb)


def _edge_body(m_hbm, srcb_hbm, dstb_hbm, zrows_hbm, out_hbm,
               sidx0, sidx1, sidx2, didx0, didx1, didx2,
               buf0, buf1, buf2, acc,
               gsem0, gsem1, gsem2, isem0, isem1, isem2,
               dsem0, dsem1, dsem2):
  c = lax.axis_index("c")
  s = lax.axis_index("s")
  wid = c * NS + s
  rbase = s * RPT
  last = s == NS - 1

  @pl.when(jnp.logical_and(c == 0, jnp.logical_not(last)))
  def _():
    pltpu.sync_copy(m_hbm.at[pl.ds(rbase, RPT)], acc.at[pl.ds(rbase, RPT)])

  @pl.when(jnp.logical_and(c == 0, last))
  def _():
    pltpu.sync_copy(m_hbm.at[pl.ds((NS - 1) * RPT, RPT_LAST)],
                    acc.at[pl.ds((NS - 1) * RPT, RPT_LAST)])

  @pl.when(jnp.logical_and(c != 0, jnp.logical_not(last)))
  def _():
    pltpu.sync_copy(zrows_hbm.at[pl.ds(0, RPT)], acc.at[pl.ds(rbase, RPT)])

  @pl.when(jnp.logical_and(c != 0, last))
  def _():
    pltpu.sync_copy(zrows_hbm, acc.at[pl.ds((NS - 1) * RPT, RPT_LAST)])

  plsc.subcore_barrier()

  sidx = (sidx0, sidx1, sidx2)
  didx = (didx0, didx1, didx2)
  bufs = (buf0, buf1, buf2)
  gsem = (gsem0, gsem1, gsem2)
  isem = (isem0, isem1, isem2)
  dsem = (dsem0, dsem1, dsem2)

  def wait_sidx(j):
    pltpu.make_async_copy(srcb_hbm.at[0, 0], sidx[j], isem[j]).wait()

  def wait_didx(j):
    pltpu.make_async_copy(dstb_hbm.at[0, 0], didx[j], dsem[j]).wait()

  def wait_gather(j):
    pltpu.make_async_copy(m_hbm.at[sidx0.at[0]], bufs[j], gsem[j]).wait()

  for j in range(3):
    pltpu.async_copy(srcb_hbm.at[wid, j], sidx[j], isem[j])
    pltpu.async_copy(dstb_hbm.at[wid, j], didx[j], dsem[j])
  for j in range(3):
    wait_sidx(j)
    pltpu.async_copy(m_hbm.at[sidx[j].at[0]], bufs[j], gsem[j])

  @pl.loop(0, NB // 3)
  def _groups(o):
    B = 3 * o
    for j in range(3):
      b = B + j
      wait_gather(j)

      @pl.when(b + 3 < NB)
      def _():
        pltpu.async_copy(srcb_hbm.at[wid, b + 3], sidx[j], isem[j])

      wait_didx(j)
      pltpu.sync_copy(bufs[j], acc.at[didx[j].at[0]], add=True)

      @pl.when(b + 3 < NB)
      def _():
        pltpu.async_copy(dstb_hbm.at[wid, b + 3], didx[j], dsem[j])
        wait_sidx(j)
        pltpu.async_copy(m_hbm.at[sidx[j].at[0]], bufs[j], gsem[j])

  plsc.subcore_barrier()

  @pl.when(jnp.logical_not(last))
  def _():
    pltpu.sync_copy(acc.at[pl.ds(rbase, RPT)],
                    out_hbm.at[c, pl.ds(rbase, RPT)])

  @pl.when(last)
  def _():
    pltpu.sync_copy(acc.at[pl.ds((NS - 1) * RPT, RPT_LAST)],
                    out_hbm.at[c, pl.ds((NS - 1) * RPT, RPT_LAST)])


def _edge_call(m, srcb4, dstb4, zrows):
  f = pl.kernel(
      _edge_body,
      out_type=jax.ShapeDtypeStruct((NC, N, D), jnp.float32),
      mesh=_mesh(),
      scratch_types=(
          [pltpu.VMEM((1, K), jnp.int32)] * 6
          + [pltpu.VMEM((K, D), jnp.float32)] * 3
          + [pltpu.VMEM_SHARED((NACC, D), jnp.float32)]
          + [pltpu.SemaphoreType.DMA] * 9
      ),
  )
  return f(m, srcb4, dstb4, zrows)


RB = 1000


def _tc_first_body(x_ref, wx_ref, bx_ref, w0_ref, dinv_ref, o_ref):
  h0 = jnp.dot(x_ref[...], wx_ref[...],
               preferred_element_type=jnp.float32) + bx_ref[...]
  o_ref[...] = jnp.dot(h0, w0_ref[...],
                       preferred_element_type=jnp.float32) * dinv_ref[...]


def _tc_mid_body(s_ref, b_ref, w_ref, dinv_ref, o_ref):
  h = jnp.maximum(dinv_ref[...] * (s_ref[0] + s_ref[1]) + b_ref[...], 0.0)
  o_ref[...] = jnp.dot(h, w_ref[...],
                       preferred_element_type=jnp.float32) * dinv_ref[...]


def _tc_last_body(s_ref, b_ref, dinv_ref, o_ref):
  o_ref[...] = dinv_ref[...] * (s_ref[0] + s_ref[1]) + b_ref[...]


_row_spec = pl.BlockSpec((RB, D), lambda i: (i, 0))
_w_spec = pl.BlockSpec((D, D), lambda i: (0, 0))
_b_spec = pl.BlockSpec((1, D), lambda i: (0, 0))
_dinv_spec = pl.BlockSpec((RB, 1), lambda i: (i, 0))
_s_spec = pl.BlockSpec((NC, RB, D), lambda i: (0, i, 0))
_out_sds = jax.ShapeDtypeStruct((N, D), jnp.float32)


def _tc_first(x, Wx, bx2, W0, dinv):
  return pl.pallas_call(
      _tc_first_body, grid=(N // RB,),
      in_specs=[_row_spec, _w_spec, _b_spec, _w_spec, _dinv_spec],
      out_specs=_row_spec, out_shape=_out_sds,
  )(x, Wx, bx2, W0, dinv)


def _tc_mid(s2, b2, W, dinv):
  return pl.pallas_call(
      _tc_mid_body, grid=(N // RB,),
      in_specs=[_s_spec, _b_spec, _w_spec, _dinv_spec],
      out_specs=_row_spec, out_shape=_out_sds,
  )(s2, b2, W, dinv)


def _tc_last(s2, b2, dinv):
  return pl.pallas_call(
      _tc_last_body, grid=(N // RB,),
      in_specs=[_s_spec, _b_spec, _dinv_spec],
      out_specs=_row_spec, out_shape=_out_sds,
  )(s2, b2, dinv)


def kernel(x, edge_index, edge_attr, Wx, bx, W0, b0, W1, b1, W2, b2):
  del edge_attr
  src = edge_index[0].astype(jnp.int32)
  dst = edge_index[1].astype(jnp.int32)
  pad = EPAD - E
  ar = jnp.arange(pad, dtype=jnp.int32)
  pad_src = (ar * 97) % N
  pad_dst = N + ar % (NACC - N)
  srcb4 = jnp.concatenate([src, pad_src]).reshape(NW, NB, 1, K)
  dstb4 = jnp.concatenate([dst, pad_dst]).reshape(NW, NB, 1, K)
  zrows = jnp.zeros((RPT_LAST, D), jnp.float32)

  deg2 = _deg_call(dstb4.reshape(NW, NB, K))
  deg = deg2[0, 0, :N] + deg2[1, 0, :N] + 1.0
  dinv = lax.rsqrt(deg).reshape(N, 1)

  bx2 = bx.reshape(1, D)
  m = _tc_first(x, Wx, bx2, W0, dinv)
  s2 = _edge_call(m, srcb4, dstb4, zrows)
  m = _tc_mid(s2, b0.reshape(1, D), W1, dinv)
  s2 = _edge_call(m, srcb4, dstb4, zrows)
  m = _tc_mid(s2, b1.reshape(1, D), W2, dinv)
  s2 = _edge_call(m, srcb4, dstb4, zrows)
  return _tc_last(s2, b2.reshape(1, D), dinv)
